# baseline (speedup 1.0000x reference)
import jax
import jax.numpy as jnp
from jax import lax
from jax.experimental import pallas as pl
from jax.experimental.pallas import tpu as pltpu
from jax.experimental.pallas import tpu_sc as plsc

N_NODES = 10000
NPAD = 10240
T = 16
ROWS_PER_TILE = NPAD // T
CHUNK = 128
QUARTERS = 4
QLEN = 5120
SHARD = QUARTERS * QLEN
EDGES_PAD = T * SHARD
KS = 5
NSUB = 1000
SUB_PAD = 1024
SUB_CH = SUB_PAD // CHUNK
VCHUNKS = ROWS_PER_TILE // 16
LN2 = 0.6931471805599453


def _ln(x):
    xi = plsc.bitcast(x, jnp.int32)
    e = lax.shift_right_arithmetic(xi, 23) - 127
    m = plsc.bitcast(
        lax.bitwise_or(lax.bitwise_and(xi, 0x007FFFFF), 0x3F800000), jnp.float32
    )
    s = (m - 1.0) / (m + 1.0)
    s2 = s * s
    poly = 1.0 + s2 * (1.0 / 3.0 + s2 * (1.0 / 5.0 + s2 * (1.0 / 7.0 + s2 * (1.0 / 9.0))))
    return e.astype(jnp.float32) * LN2 + 2.0 * s * poly


def _body(rows_hbm, cols_hbm, sub_hbm, alpha_hbm,
          e_hbm, loss_hbm,
          r0, r1, r2, r3, c0, c1, c2, c3, g0, g1, g2, g3, eold, nbuf, dinv_v, zb,
          ones_q, ones_c, lbuf, subv, alv, ne_sh, e_sh,
          sg0, sg1, sg2, sg3, ss0, ss1, ss2, ss3):
    c = lax.axis_index("c")
    t = lax.axis_index("s")
    base = t * ROWS_PER_TILE
    ebase = c * NPAD + base
    my_rows = pl.ds(base, ROWS_PER_TILE)
    my_erows = pl.ds(ebase, ROWS_PER_TILE)
    rqs = (r0, r1, r2, r3)
    cqs = (c0, c1, c2, c3)

    gsems4 = (sg0, sg1, sg2, sg3)
    ssems4 = (ss0, ss1, ss2, ss3)
    rdesc = [pltpu.async_copy(rows_hbm.at[t * QUARTERS + q], rqs[q], gsems4[q])
             for q in range(QUARTERS)]
    cdesc = [pltpu.async_copy(cols_hbm.at[t * QUARTERS + q], cqs[q], ssems4[q])
             for q in range(QUARTERS)]

    zeros16 = jnp.zeros((16,), jnp.float32)
    ones16 = jnp.ones((16,), jnp.float32)
    for i in range(VCHUNKS):
        zb[pl.ds(i * 16, 16)] = zeros16
    for i in range(QLEN // 16):
        ones_q[pl.ds(i * 16, 16)] = ones16
    for i in range(CHUNK // 16):
        ones_c[pl.ds(i * 16, 16)] = ones16
    for d in rdesc:
        d.wait()
    for d in cdesc:
        d.wait()
    pltpu.sync_copy(sub_hbm.at[pl.ds(c * SUB_CH, SUB_CH)], subv)
    pltpu.sync_copy(alpha_hbm, alv)

    pltpu.sync_copy(zb, e_sh.at[my_rows])
    pltpu.sync_copy(zb, ne_sh.at[my_rows])
    plsc.subcore_barrier()

    @pl.when(t < SUB_CH)
    def _():
        pltpu.sync_copy(ones_c, e_sh.at[subv.at[t]])

    for q in range(QUARTERS):
        pltpu.async_copy(ones_q, ne_sh.at[rqs[q]], ssems4[q], add=True)
    for q in range(QUARTERS):
        pltpu.make_async_copy(ones_q, ne_sh.at[rqs[q]], ssems4[q]).wait()

    plsc.subcore_barrier()
    pltpu.sync_copy(ne_sh.at[my_rows], nbuf)
    pltpu.sync_copy(zb, ne_sh.at[my_rows])
    for i in range(VCHUNKS):
        sl = pl.ds(i * 16, 16)
        dinv_v[sl] = 1.0 / jnp.maximum(nbuf[sl], 1e-12)

    a = alv[...]
    alpha = 1.0 / (1.0 + jnp.exp(-a))
    one_m_alpha = 1.0 - alpha
    plsc.subcore_barrier()

    @pl.loop(0, KS)
    def _(s):
        gbufs = (g0, g1, g2, g3)
        for q in range(QUARTERS):
            pltpu.async_copy(e_sh.at[cqs[q]], gbufs[q], gsems4[q])
        for q in range(QUARTERS):
            pltpu.make_async_copy(
                e_sh.at[cqs[q]], gbufs[q], gsems4[q]).wait()
            pltpu.async_copy(
                gbufs[q], ne_sh.at[rqs[q]], ssems4[q], add=True)
        for q in range(QUARTERS):
            pltpu.make_async_copy(
                gbufs[q], ne_sh.at[rqs[q]], ssems4[q]).wait()

        plsc.subcore_barrier()

        nd = pltpu.async_copy(ne_sh.at[my_rows], nbuf, sg0)
        ed = pltpu.async_copy(e_sh.at[my_rows], eold, sg1)
        nd.wait()
        zd = pltpu.async_copy(zb, ne_sh.at[my_rows], sg2)
        ed.wait()
        for i in range(VCHUNKS):
            sl = pl.ds(i * 16, 16)
            eold[sl] = alpha * eold[sl] + one_m_alpha * dinv_v[sl] * nbuf[sl]
        pltpu.sync_copy(eold, e_sh.at[my_rows])
        zd.wait()

        @pl.when(s == KS - 1)
        def _():
            pltpu.sync_copy(eold, e_hbm.at[my_erows])

        plsc.subcore_barrier()

    @pl.when(t == 0)
    def _():
        iot = lax.iota(jnp.int32, 16)
        acc = jnp.zeros((16,), jnp.float32)
        for j in range(SUB_CH):
            pltpu.sync_copy(e_sh.at[subv.at[j]], lbuf)
            for i in range(CHUNK // 16):
                gidx = j * CHUNK + i * 16 + iot
                p = jnp.maximum(lbuf[pl.ds(i * 16, 16)], 1e-6)
                acc = acc + jnp.where(gidx < NSUB, _ln(p), 0.0)
        total = jnp.sum(acc * (-1.0 / NSUB))
        alv[...] = jnp.broadcast_to(total, (16,))
        pltpu.sync_copy(alv, loss_hbm.at[c])


def kernel(embeddings, edge_index, sub_pos, sub_neg, raw_alpha):
    del embeddings
    row = edge_index[0]
    col = edge_index[1]
    pad_e = EDGES_PAD - row.shape[0]
    pad_idx = jnp.full((pad_e,), NPAD - 1, jnp.int32)
    rows = jnp.concatenate([row, pad_idx]).reshape(T * QUARTERS, QLEN)
    cols = jnp.concatenate([col, pad_idx]).reshape(T * QUARTERS, QLEN)
    pad_s = jnp.full((SUB_PAD - NSUB,), NPAD - 16, jnp.int32)
    sub = jnp.concatenate([
        jnp.concatenate([sub_neg, pad_s]),
        jnp.concatenate([sub_pos, pad_s]),
    ]).reshape(2 * SUB_CH, CHUNK)
    al = jnp.broadcast_to(raw_alpha.astype(jnp.float32), (16,))

    mesh = plsc.VectorSubcoreMesh(core_axis_name="c", subcore_axis_name="s")
    f32 = jnp.float32
    fn = pl.kernel(
        _body,
        compiler_params=pltpu.CompilerParams(needs_layout_passes=False),
        out_type=[
            jax.ShapeDtypeStruct((2 * NPAD,), f32),
            jax.ShapeDtypeStruct((2, 16), f32),
        ],
        mesh=mesh,
        scratch_types=[
            pltpu.VMEM((QLEN,), jnp.int32),
            pltpu.VMEM((QLEN,), jnp.int32),
            pltpu.VMEM((QLEN,), jnp.int32),
            pltpu.VMEM((QLEN,), jnp.int32),
            pltpu.VMEM((QLEN,), jnp.int32),
            pltpu.VMEM((QLEN,), jnp.int32),
            pltpu.VMEM((QLEN,), jnp.int32),
            pltpu.VMEM((QLEN,), jnp.int32),
            pltpu.VMEM((QLEN,), f32),
            pltpu.VMEM((QLEN,), f32),
            pltpu.VMEM((QLEN,), f32),
            pltpu.VMEM((QLEN,), f32),
            pltpu.VMEM((ROWS_PER_TILE,), f32),
            pltpu.VMEM((ROWS_PER_TILE,), f32),
            pltpu.VMEM((ROWS_PER_TILE,), f32),
            pltpu.VMEM((ROWS_PER_TILE,), f32),
            pltpu.VMEM((QLEN,), f32),
            pltpu.VMEM((CHUNK,), f32),
            pltpu.VMEM((CHUNK,), f32),
            pltpu.VMEM((SUB_CH, CHUNK), jnp.int32),
            pltpu.VMEM((16,), f32),
            pltpu.VMEM_SHARED((NPAD,), f32),
            pltpu.VMEM_SHARED((NPAD,), f32),
            pltpu.SemaphoreType.DMA,
            pltpu.SemaphoreType.DMA,
            pltpu.SemaphoreType.DMA,
            pltpu.SemaphoreType.DMA,
            pltpu.SemaphoreType.DMA,
            pltpu.SemaphoreType.DMA,
            pltpu.SemaphoreType.DMA,
            pltpu.SemaphoreType.DMA,
        ],
    )
    e, lv = fn(rows, cols, sub, al)
    E = jnp.stack([e[:N_NODES], e[NPAD:NPAD + N_NODES]], axis=1)
    return (lv[0, 0] + lv[1, 0], E)


# speedup vs baseline: 1.5695x; 1.5695x over previous
import jax
import jax.numpy as jnp
from jax import lax
from jax.experimental import pallas as pl
from jax.experimental.pallas import tpu as pltpu
from jax.experimental.pallas import tpu_sc as plsc

N_NODES = 10000
NPAD = 10240
T = 16
ROWS_PER_TILE = NPAD // T
CHUNK = 128
QUARTERS = 4
QLEN = 5120
BLK = 1280
BPQ = QLEN // BLK
SHARD = QUARTERS * QLEN
EDGES_PAD = T * SHARD
KS = 5
NSUB = 1000
SUB_PAD = 1024
SUB_CH = SUB_PAD // CHUNK
VCHUNKS = ROWS_PER_TILE // 16
LN2 = 0.6931471805599453


def _ln(x):
    xi = plsc.bitcast(x, jnp.int32)
    e = lax.shift_right_arithmetic(xi, 23) - 127
    m = plsc.bitcast(
        lax.bitwise_or(lax.bitwise_and(xi, 0x007FFFFF), 0x3F800000), jnp.float32
    )
    s = (m - 1.0) / (m + 1.0)
    s2 = s * s
    poly = 1.0 + s2 * (1.0 / 3.0 + s2 * (1.0 / 5.0 + s2 * (1.0 / 7.0 + s2 * (1.0 / 9.0))))
    return e.astype(jnp.float32) * LN2 + 2.0 * s * poly


def _body(rows_hbm, cols_hbm, sub_hbm, alpha_hbm,
          e_hbm, loss_hbm,
          r0, r1, r2, r3, c0, c1, c2, c3, g0, g1, g2, g3, eold, nbuf, dinv_v, zb,
          ones_q, ones_c, lbuf, subv, alv, ne_sh, e_sh,
          sg0, sg1, sg2, sg3, ss0, ss1, ss2, ss3):
    c = lax.axis_index("c")
    t = lax.axis_index("s")
    base = t * ROWS_PER_TILE
    ebase = c * NPAD + base
    my_rows = pl.ds(base, ROWS_PER_TILE)
    my_erows = pl.ds(ebase, ROWS_PER_TILE)
    rqs = (r0, r1, r2, r3)
    cqs = (c0, c1, c2, c3)

    gsems4 = (sg0, sg1, sg2, sg3)
    ssems4 = (ss0, ss1, ss2, ss3)
    rdesc = [pltpu.async_copy(rows_hbm.at[t * QUARTERS + q], rqs[q], gsems4[q])
             for q in range(QUARTERS)]
    cdesc = [pltpu.async_copy(cols_hbm.at[t * QUARTERS + q], cqs[q], ssems4[q])
             for q in range(QUARTERS)]

    zeros16 = jnp.zeros((16,), jnp.float32)
    ones16 = jnp.ones((16,), jnp.float32)
    for i in range(VCHUNKS):
        zb[pl.ds(i * 16, 16)] = zeros16
    for i in range(QLEN // 16):
        ones_q[pl.ds(i * 16, 16)] = ones16
    for i in range(CHUNK // 16):
        ones_c[pl.ds(i * 16, 16)] = ones16
    for d in rdesc:
        d.wait()
    for d in cdesc:
        d.wait()
    pltpu.sync_copy(sub_hbm.at[pl.ds(c * SUB_CH, SUB_CH)], subv)
    pltpu.sync_copy(alpha_hbm, alv)

    pltpu.sync_copy(zb, e_sh.at[my_rows])
    pltpu.sync_copy(zb, ne_sh.at[my_rows])
    plsc.subcore_barrier()

    @pl.when(t < SUB_CH)
    def _():
        pltpu.sync_copy(ones_c, e_sh.at[subv.at[t]])

    for q in range(QUARTERS):
        pltpu.async_copy(ones_q, ne_sh.at[rqs[q]], ssems4[q], add=True)
    for q in range(QUARTERS):
        pltpu.make_async_copy(ones_q, ne_sh.at[rqs[q]], ssems4[q]).wait()

    plsc.subcore_barrier()
    pltpu.sync_copy(ne_sh.at[my_rows], nbuf)
    pltpu.sync_copy(zb, ne_sh.at[my_rows])
    for i in range(VCHUNKS):
        sl = pl.ds(i * 16, 16)
        dinv_v[sl] = 1.0 / jnp.maximum(nbuf[sl], 1e-12)

    a = alv[...]
    alpha = 1.0 / (1.0 + jnp.exp(-a))
    one_m_alpha = 1.0 - alpha
    plsc.subcore_barrier()

    @pl.loop(0, KS)
    def _(s):
        gbufs = (g0, g1, g2, g3)

        def cref(k):
            return cqs[k // BPQ].at[pl.ds((k % BPQ) * BLK, BLK)]

        def rref(k):
            return rqs[k // BPQ].at[pl.ds((k % BPQ) * BLK, BLK)]

        nblk = QUARTERS * BPQ
        for b in range(4):
            pltpu.async_copy(e_sh.at[cref(b)], gbufs[b], gsems4[b])
        for j in range(0, nblk, 4):
            for b in range(4):
                k = j + b
                pltpu.make_async_copy(
                    e_sh.at[cref(k)], gbufs[b], gsems4[b]).wait()
                pltpu.async_copy(
                    gbufs[b], ne_sh.at[rref(k)], ssems4[b], add=True)
            for b in range(4):
                k = j + b
                if k + 4 < nblk:
                    pltpu.make_async_copy(
                        gbufs[b], ne_sh.at[rref(k)], ssems4[b]).wait()
                    pltpu.async_copy(
                        e_sh.at[cref(k + 4)], gbufs[b], gsems4[b])
        for b in range(4):
            k = nblk - 4 + b
            pltpu.make_async_copy(
                gbufs[b], ne_sh.at[rref(k)], ssems4[b]).wait()

        plsc.subcore_barrier()

        nd = pltpu.async_copy(ne_sh.at[my_rows], nbuf, sg0)
        ed = pltpu.async_copy(e_sh.at[my_rows], eold, sg1)
        nd.wait()
        zd = pltpu.async_copy(zb, ne_sh.at[my_rows], sg2)
        ed.wait()
        for i in range(VCHUNKS):
            sl = pl.ds(i * 16, 16)
            eold[sl] = alpha * eold[sl] + one_m_alpha * dinv_v[sl] * nbuf[sl]
        pltpu.sync_copy(eold, e_sh.at[my_rows])
        zd.wait()

        @pl.when(s == KS - 1)
        def _():
            pltpu.sync_copy(eold, e_hbm.at[my_erows])

        plsc.subcore_barrier()

    @pl.when(t == 0)
    def _():
        iot = lax.iota(jnp.int32, 16)
        acc = jnp.zeros((16,), jnp.float32)
        for j in range(SUB_CH):
            pltpu.sync_copy(e_sh.at[subv.at[j]], lbuf)
            for i in range(CHUNK // 16):
                gidx = j * CHUNK + i * 16 + iot
                p = jnp.maximum(lbuf[pl.ds(i * 16, 16)], 1e-6)
                acc = acc + jnp.where(gidx < NSUB, _ln(p), 0.0)
        total = jnp.sum(acc * (-1.0 / NSUB))
        alv[...] = jnp.broadcast_to(total, (16,))
        pltpu.sync_copy(alv, loss_hbm.at[c])


def kernel(embeddings, edge_index, sub_pos, sub_neg, raw_alpha):
    del embeddings
    row = edge_index[0]
    col = edge_index[1]
    pad_e = EDGES_PAD - row.shape[0]
    pad_idx = (N_NODES + (jnp.arange(pad_e, dtype=jnp.int32) % 224))
    rows = jnp.concatenate([row, pad_idx]).reshape(T * QUARTERS, QLEN)
    cols = jnp.concatenate([col, pad_idx]).reshape(T * QUARTERS, QLEN)
    pad_s = jnp.full((SUB_PAD - NSUB,), NPAD - 16, jnp.int32)
    sub = jnp.concatenate([
        jnp.concatenate([sub_neg, pad_s]),
        jnp.concatenate([sub_pos, pad_s]),
    ]).reshape(2 * SUB_CH, CHUNK)
    al = jnp.broadcast_to(raw_alpha.astype(jnp.float32), (16,))

    mesh = plsc.VectorSubcoreMesh(core_axis_name="c", subcore_axis_name="s")
    f32 = jnp.float32
    fn = pl.kernel(
        _body,
        compiler_params=pltpu.CompilerParams(needs_layout_passes=False),
        out_type=[
            jax.ShapeDtypeStruct((2 * NPAD,), f32),
            jax.ShapeDtypeStruct((2, 16), f32),
        ],
        mesh=mesh,
        scratch_types=[
            pltpu.VMEM((QLEN,), jnp.int32),
            pltpu.VMEM((QLEN,), jnp.int32),
            pltpu.VMEM((QLEN,), jnp.int32),
            pltpu.VMEM((QLEN,), jnp.int32),
            pltpu.VMEM((QLEN,), jnp.int32),
            pltpu.VMEM((QLEN,), jnp.int32),
            pltpu.VMEM((QLEN,), jnp.int32),
            pltpu.VMEM((QLEN,), jnp.int32),
            pltpu.VMEM((BLK,), f32),
            pltpu.VMEM((BLK,), f32),
            pltpu.VMEM((BLK,), f32),
            pltpu.VMEM((BLK,), f32),
            pltpu.VMEM((ROWS_PER_TILE,), f32),
            pltpu.VMEM((ROWS_PER_TILE,), f32),
            pltpu.VMEM((ROWS_PER_TILE,), f32),
            pltpu.VMEM((ROWS_PER_TILE,), f32),
            pltpu.VMEM((QLEN,), f32),
            pltpu.VMEM((CHUNK,), f32),
            pltpu.VMEM((CHUNK,), f32),
            pltpu.VMEM((SUB_CH, CHUNK), jnp.int32),
            pltpu.VMEM((16,), f32),
            pltpu.VMEM_SHARED((NPAD,), f32),
            pltpu.VMEM_SHARED((NPAD,), f32),
            pltpu.SemaphoreType.DMA,
            pltpu.SemaphoreType.DMA,
            pltpu.SemaphoreType.DMA,
            pltpu.SemaphoreType.DMA,
            pltpu.SemaphoreType.DMA,
            pltpu.SemaphoreType.DMA,
            pltpu.SemaphoreType.DMA,
            pltpu.SemaphoreType.DMA,
        ],
    )
    e, lv = fn(rows, cols, sub, al)
    E = jnp.stack([e[:N_NODES], e[NPAD:NPAD + N_NODES]], axis=1)
    return (lv[0, 0] + lv[1, 0], E)


# speedup vs baseline: 1.6671x; 1.0622x over previous
import jax
import jax.numpy as jnp
from jax import lax
from jax.experimental import pallas as pl
from jax.experimental.pallas import tpu as pltpu
from jax.experimental.pallas import tpu_sc as plsc

N_NODES = 10000
NPAD = 10240
T = 16
ROWS_PER_TILE = NPAD // T
CHUNK = 128
QUARTERS = 4
QLEN = 5120
BLK = 2560
BPQ = QLEN // BLK
SHARD = QUARTERS * QLEN
EDGES_PAD = T * SHARD
KS = 5
NSUB = 1000
SUB_PAD = 1024
SUB_CH = SUB_PAD // CHUNK
VCHUNKS = ROWS_PER_TILE // 16
LN2 = 0.6931471805599453


def _ln(x):
    xi = plsc.bitcast(x, jnp.int32)
    e = lax.shift_right_arithmetic(xi, 23) - 127
    m = plsc.bitcast(
        lax.bitwise_or(lax.bitwise_and(xi, 0x007FFFFF), 0x3F800000), jnp.float32
    )
    s = (m - 1.0) / (m + 1.0)
    s2 = s * s
    poly = 1.0 + s2 * (1.0 / 3.0 + s2 * (1.0 / 5.0 + s2 * (1.0 / 7.0 + s2 * (1.0 / 9.0))))
    return e.astype(jnp.float32) * LN2 + 2.0 * s * poly


def _body(rows_hbm, cols_hbm, sub_hbm, alpha_hbm,
          e_hbm, loss_hbm,
          r0, r1, r2, r3, c0, c1, c2, c3, g0, g1, g2, g3, eold, nbuf, dinv_v, zb,
          ones_q, ones_c, lbuf, subv, alv, ne_sh, e_sh,
          sg0, sg1, sg2, sg3, ss0, ss1, ss2, ss3):
    c = lax.axis_index("c")
    t = lax.axis_index("s")
    base = t * ROWS_PER_TILE
    ebase = c * NPAD + base
    my_rows = pl.ds(base, ROWS_PER_TILE)
    my_erows = pl.ds(ebase, ROWS_PER_TILE)
    rqs = (r0, r1, r2, r3)
    cqs = (c0, c1, c2, c3)

    gsems4 = (sg0, sg1, sg2, sg3)
    ssems4 = (ss0, ss1, ss2, ss3)
    rdesc = [pltpu.async_copy(rows_hbm.at[t * QUARTERS + q], rqs[q], gsems4[q])
             for q in range(QUARTERS)]
    cdesc = [pltpu.async_copy(cols_hbm.at[t * QUARTERS + q], cqs[q], ssems4[q])
             for q in range(QUARTERS)]

    zeros16 = jnp.zeros((16,), jnp.float32)
    ones16 = jnp.ones((16,), jnp.float32)
    for i in range(VCHUNKS):
        zb[pl.ds(i * 16, 16)] = zeros16
    for i in range(QLEN // 16):
        ones_q[pl.ds(i * 16, 16)] = ones16
    for i in range(CHUNK // 16):
        ones_c[pl.ds(i * 16, 16)] = ones16
    for d in rdesc:
        d.wait()
    for d in cdesc:
        d.wait()
    pltpu.sync_copy(sub_hbm.at[pl.ds(c * SUB_CH, SUB_CH)], subv)
    pltpu.sync_copy(alpha_hbm, alv)

    pltpu.sync_copy(zb, e_sh.at[my_rows])
    pltpu.sync_copy(zb, ne_sh.at[my_rows])
    plsc.subcore_barrier()

    @pl.when(t < SUB_CH)
    def _():
        pltpu.sync_copy(ones_c, e_sh.at[subv.at[t]])

    for q in range(QUARTERS):
        pltpu.async_copy(ones_q, ne_sh.at[rqs[q]], ssems4[q], add=True)
    for q in range(QUARTERS):
        pltpu.make_async_copy(ones_q, ne_sh.at[rqs[q]], ssems4[q]).wait()

    plsc.subcore_barrier()
    pltpu.sync_copy(ne_sh.at[my_rows], nbuf)
    pltpu.sync_copy(zb, ne_sh.at[my_rows])
    for i in range(VCHUNKS):
        sl = pl.ds(i * 16, 16)
        dinv_v[sl] = 1.0 / jnp.maximum(nbuf[sl], 1e-12)

    a = alv[...]
    alpha = 1.0 / (1.0 + jnp.exp(-a))
    one_m_alpha = 1.0 - alpha
    plsc.subcore_barrier()

    @pl.loop(0, KS)
    def _(s):
        gbufs = (g0, g1, g2, g3)

        def cref(k):
            return cqs[k // BPQ].at[pl.ds((k % BPQ) * BLK, BLK)]

        def rref(k):
            return rqs[k // BPQ].at[pl.ds((k % BPQ) * BLK, BLK)]

        nblk = QUARTERS * BPQ
        for b in range(4):
            pltpu.async_copy(e_sh.at[cref(b)], gbufs[b], gsems4[b])
        for j in range(0, nblk, 4):
            for b in range(4):
                k = j + b
                pltpu.make_async_copy(
                    e_sh.at[cref(k)], gbufs[b], gsems4[b]).wait()
                pltpu.async_copy(
                    gbufs[b], ne_sh.at[rref(k)], ssems4[b], add=True)
            for b in range(4):
                k = j + b
                if k + 4 < nblk:
                    pltpu.make_async_copy(
                        gbufs[b], ne_sh.at[rref(k)], ssems4[b]).wait()
                    pltpu.async_copy(
                        e_sh.at[cref(k + 4)], gbufs[b], gsems4[b])
        for b in range(4):
            k = nblk - 4 + b
            pltpu.make_async_copy(
                gbufs[b], ne_sh.at[rref(k)], ssems4[b]).wait()

        plsc.subcore_barrier()

        nd = pltpu.async_copy(ne_sh.at[my_rows], nbuf, sg0)
        ed = pltpu.async_copy(e_sh.at[my_rows], eold, sg1)
        nd.wait()
        zd = pltpu.async_copy(zb, ne_sh.at[my_rows], sg2)
        ed.wait()
        for i in range(VCHUNKS):
            sl = pl.ds(i * 16, 16)
            eold[sl] = alpha * eold[sl] + one_m_alpha * dinv_v[sl] * nbuf[sl]
        pltpu.sync_copy(eold, e_sh.at[my_rows])
        zd.wait()

        @pl.when(s == KS - 1)
        def _():
            pltpu.sync_copy(eold, e_hbm.at[my_erows])

        plsc.subcore_barrier()

    @pl.when(t == 0)
    def _():
        iot = lax.iota(jnp.int32, 16)
        acc = jnp.zeros((16,), jnp.float32)
        for j in range(SUB_CH):
            pltpu.sync_copy(e_sh.at[subv.at[j]], lbuf)
            for i in range(CHUNK // 16):
                gidx = j * CHUNK + i * 16 + iot
                p = jnp.maximum(lbuf[pl.ds(i * 16, 16)], 1e-6)
                acc = acc + jnp.where(gidx < NSUB, _ln(p), 0.0)
        total = jnp.sum(acc * (-1.0 / NSUB))
        alv[...] = jnp.broadcast_to(total, (16,))
        pltpu.sync_copy(alv, loss_hbm.at[c])


def kernel(embeddings, edge_index, sub_pos, sub_neg, raw_alpha):
    del embeddings
    row = edge_index[0]
    col = edge_index[1]
    pad_e = EDGES_PAD - row.shape[0]
    pad_idx = (N_NODES + (jnp.arange(pad_e, dtype=jnp.int32) % 224))
    rows = jnp.concatenate([row, pad_idx]).reshape(T * QUARTERS, QLEN)
    cols = jnp.concatenate([col, pad_idx]).reshape(T * QUARTERS, QLEN)
    pad_s = jnp.full((SUB_PAD - NSUB,), NPAD - 16, jnp.int32)
    sub = jnp.concatenate([
        jnp.concatenate([sub_neg, pad_s]),
        jnp.concatenate([sub_pos, pad_s]),
    ]).reshape(2 * SUB_CH, CHUNK)
    al = jnp.broadcast_to(raw_alpha.astype(jnp.float32), (16,))

    mesh = plsc.VectorSubcoreMesh(core_axis_name="c", subcore_axis_name="s")
    f32 = jnp.float32
    fn = pl.kernel(
        _body,
        compiler_params=pltpu.CompilerParams(needs_layout_passes=False),
        out_type=[
            jax.ShapeDtypeStruct((2 * NPAD,), f32),
            jax.ShapeDtypeStruct((2, 16), f32),
        ],
        mesh=mesh,
        scratch_types=[
            pltpu.VMEM((QLEN,), jnp.int32),
            pltpu.VMEM((QLEN,), jnp.int32),
            pltpu.VMEM((QLEN,), jnp.int32),
            pltpu.VMEM((QLEN,), jnp.int32),
            pltpu.VMEM((QLEN,), jnp.int32),
            pltpu.VMEM((QLEN,), jnp.int32),
            pltpu.VMEM((QLEN,), jnp.int32),
            pltpu.VMEM((QLEN,), jnp.int32),
            pltpu.VMEM((BLK,), f32),
            pltpu.VMEM((BLK,), f32),
            pltpu.VMEM((BLK,), f32),
            pltpu.VMEM((BLK,), f32),
            pltpu.VMEM((ROWS_PER_TILE,), f32),
            pltpu.VMEM((ROWS_PER_TILE,), f32),
            pltpu.VMEM((ROWS_PER_TILE,), f32),
            pltpu.VMEM((ROWS_PER_TILE,), f32),
            pltpu.VMEM((QLEN,), f32),
            pltpu.VMEM((CHUNK,), f32),
            pltpu.VMEM((CHUNK,), f32),
            pltpu.VMEM((SUB_CH, CHUNK), jnp.int32),
            pltpu.VMEM((16,), f32),
            pltpu.VMEM_SHARED((NPAD,), f32),
            pltpu.VMEM_SHARED((NPAD,), f32),
            pltpu.SemaphoreType.DMA,
            pltpu.SemaphoreType.DMA,
            pltpu.SemaphoreType.DMA,
            pltpu.SemaphoreType.DMA,
            pltpu.SemaphoreType.DMA,
            pltpu.SemaphoreType.DMA,
            pltpu.SemaphoreType.DMA,
            pltpu.SemaphoreType.DMA,
        ],
    )
    e, lv = fn(rows, cols, sub, al)
    E = jnp.stack([e[:N_NODES], e[NPAD:NPAD + N_NODES]], axis=1)
    return (lv[0, 0] + lv[1, 0], E)
